# baseline (speedup 1.0000x reference)
import functools

import numpy as np
import jax
import jax.numpy as jnp
from jax import lax
from jax.experimental import pallas as pl
from jax.experimental.pallas import tpu as pltpu

_CD = jnp.bfloat16


def _block_kernel(cat_ref, wall_ref, mc1_ref, ba1_ref, ba0_ref, bar1_ref,
                  bar2_ref, bd4_ref, b11_ref, o_ref, *, H, C, NC, Wn, Win):
    f32 = jnp.float32
    M = o_ref.shape[0]
    Ch = C // 2
    N = o_ref.shape[1]

    rows = lax.broadcasted_iota(jnp.int32, (M, 1), 0) % H
    keep_top = (rows != 0).astype(f32)
    keep_bot = (rows != H - 1).astype(f32)

    def combine(p, n):
        return (p[:, n:2 * n]
                + pltpu.roll(p[:, :n], 1, 0) * keep_top
                + pltpu.roll(p[:, 2 * n:], M - 1, 0) * keep_bot)

    def prelu(y, ba_ref, lo):
        n = y.shape[1]
        y = y + ba_ref[0:1, lo:lo + n]
        return jnp.where(y > 0, y, ba_ref[1:2, lo:lo + n] * y)

    def lane_pad(x, w):
        z = jnp.zeros((M, w), x.dtype)
        return jnp.concatenate([z, x, z], axis=1)

    skip2d = cat_ref[:, :Wn * Ch]
    xv2d = cat_ref[:, Wn * Ch:Wn * Ch + Win * Ch]

    p = jnp.dot(xv2d, mc1_ref[...], preferred_element_type=f32)
    up = prelu(combine(p, N // 2), ba1_ref, 0).astype(_CD)

    skr = lane_pad(skip2d, Ch)
    upr = lane_pad(up, Ch)
    t_chunks = []
    for c in range(NC):
        a = jnp.dot(skr[:, 2 * Ch * c:2 * Ch * c + 4 * Ch],
                    wall_ref[c, 2 * 4 * C:2 * 4 * C + 4 * Ch],
                    preferred_element_type=f32)
        b = jnp.dot(upr[:, 2 * Ch * c:2 * Ch * c + 4 * Ch],
                    wall_ref[c, 2 * 4 * C + 4 * Ch:],
                    preferred_element_type=f32)
        t_chunks.append(prelu(combine(a + b, 2 * C), ba0_ref, 2 * C * c))
    tb = jnp.concatenate(t_chunks, axis=1).astype(_CD)

    res_chunks = [
        jnp.dot(tb[:, 4 * C * c:4 * C * (c + 1)], bd4_ref[...],
                preferred_element_type=f32)
        for c in range(NC // 2)
    ]
    res = jnp.concatenate(res_chunks, axis=1) + b11_ref[...]

    def conv_chunked(x, w_lo, ba_ref):
        xr = lane_pad(x, C)
        outs = []
        for c in range(NC):
            pc = jnp.dot(xr[:, 2 * C * c:2 * C * c + 4 * C],
                         wall_ref[c, w_lo:w_lo + 4 * C],
                         preferred_element_type=f32)
            outs.append(prelu(combine(pc, 2 * C), ba_ref, 2 * C * c))
        return outs

    y1 = jnp.concatenate(conv_chunked(tb, 0, bar1_ref), axis=1).astype(_CD)
    y2 = jnp.concatenate(conv_chunked(y1, 4 * C, bar2_ref), axis=1)

    o_ref[...] = res + y2


def _chunk_band(m, cin, cout, W, NC):
    N = W * cout
    mp = jnp.pad(m, ((cin, cin), (0, 0)))
    return jnp.stack([
        jnp.concatenate(
            [mp[2 * cin * c:2 * cin * c + 4 * cin,
                d * N + 2 * cout * c:d * N + 2 * cout * (c + 1)]
             for d in range(3)], axis=1)
        for c in range(NC)
    ])


def kernel(input_nchw, skip_nchw, mc1, ba1, m0, ba0, m1, bar1, m2, bar2,
           w11, b11):
    B, Chalf, Hin, Win = input_nchw.shape
    _, _, H, W = skip_nchw.shape
    N = b11.shape[1]
    C = N // W
    NC = W // 2

    bt = 16 if B % 16 == 0 else B
    grid = B // bt
    M = bt * H

    src_h = np.floor(np.arange(H) * (Hin / H)).astype(np.int32)
    cat = jnp.concatenate([skip_nchw, input_nchw[:, :, src_h, :]], axis=3)
    cat2d = jnp.transpose(cat, (0, 2, 3, 1)).reshape(
        B * H, (W + Win) * Chalf).astype(_CD)

    wall = jnp.concatenate([
        _chunk_band(m1, C, C, W, NC),
        _chunk_band(m2, C, C, W, NC),
        _chunk_band(m0[:W * Chalf], Chalf, C, W, NC),
        _chunk_band(m0[W * Chalf:], Chalf, C, W, NC),
    ], axis=1)

    def full(a):
        return pl.BlockSpec(a.shape, lambda b, n=a.ndim: (0,) * n)

    body = functools.partial(_block_kernel, H=H, C=C, NC=NC, Wn=W, Win=Win)

    out2d = pl.pallas_call(
        body,
        out_shape=jax.ShapeDtypeStruct((B * H, N), jnp.float32),
        grid_spec=pltpu.PrefetchScalarGridSpec(
            num_scalar_prefetch=0,
            grid=(grid,),
            in_specs=[
                pl.BlockSpec((M, (W + Win) * Chalf), lambda b: (b, 0)),
                full(wall), full(mc1), full(ba1), full(ba0),
                full(bar1), full(bar2),
                pl.BlockSpec((4 * C, 4 * C), lambda b: (0, 0)),
                full(b11),
            ],
            out_specs=pl.BlockSpec((M, N), lambda b: (b, 0)),
        ),
        compiler_params=pltpu.CompilerParams(
            dimension_semantics=("parallel",)),
    )(cat2d, wall, mc1, ba1, ba0, bar1, bar2, w11, b11)

    out = out2d.reshape(B, H, W, C)
    return jnp.transpose(out, (0, 3, 1, 2))


# speedup vs baseline: 1.1901x; 1.1901x over previous
import functools

import numpy as np
import jax
import jax.numpy as jnp
from jax import lax
from jax.experimental import pallas as pl
from jax.experimental.pallas import tpu as pltpu

_CD = jnp.bfloat16


def _block_kernel(skip_ref, xv_ref, wall_ref, mc1_ref, ba1_ref, ba0_ref,
                  bar1_ref, bar2_ref, bd4_ref, b11_ref, o_ref,
                  *, H, C, NC, Wn, Win):
    f32 = jnp.float32
    M = o_ref.shape[0]
    Ch = C // 2
    N = o_ref.shape[1]

    rows = lax.broadcasted_iota(jnp.int32, (M, 1), 0) % H
    keep_top = (rows != 0).astype(f32)
    keep_bot = (rows != H - 1).astype(f32)

    def combine(p, n):
        return (p[:, n:2 * n]
                + pltpu.roll(p[:, :n], 1, 0) * keep_top
                + pltpu.roll(p[:, 2 * n:], M - 1, 0) * keep_bot)

    def prelu(y, ba_ref, lo):
        n = y.shape[1]
        y = y + ba_ref[0:1, lo:lo + n]
        return jnp.where(y > 0, y, ba_ref[1:2, lo:lo + n] * y)

    def lane_pad(x, w):
        z = jnp.zeros((M, w), x.dtype)
        return jnp.concatenate([z, x, z], axis=1)

    skip2d = skip_ref[...]
    xv2d = xv_ref[...]

    p = jnp.dot(xv2d, mc1_ref[...], preferred_element_type=f32)
    up = prelu(combine(p, N // 2), ba1_ref, 0).astype(_CD)

    skr = lane_pad(skip2d, Ch)
    upr = lane_pad(up, Ch)
    t_chunks = []
    for c in range(NC):
        a = jnp.dot(skr[:, 2 * Ch * c:2 * Ch * c + 4 * Ch],
                    wall_ref[c, 2 * 4 * C:2 * 4 * C + 4 * Ch],
                    preferred_element_type=f32)
        b = jnp.dot(upr[:, 2 * Ch * c:2 * Ch * c + 4 * Ch],
                    wall_ref[c, 2 * 4 * C + 4 * Ch:],
                    preferred_element_type=f32)
        t_chunks.append(prelu(combine(a + b, 2 * C), ba0_ref, 2 * C * c))
    tb = jnp.concatenate(t_chunks, axis=1).astype(_CD)

    res_chunks = [
        jnp.dot(tb[:, 4 * C * c:4 * C * (c + 1)], bd4_ref[...],
                preferred_element_type=f32)
        for c in range(NC // 2)
    ]
    res = jnp.concatenate(res_chunks, axis=1) + b11_ref[...]

    def conv_chunked(x, w_lo, ba_ref):
        xr = lane_pad(x, C)
        outs = []
        for c in range(NC):
            pc = jnp.dot(xr[:, 2 * C * c:2 * C * c + 4 * C],
                         wall_ref[c, w_lo:w_lo + 4 * C],
                         preferred_element_type=f32)
            outs.append(prelu(combine(pc, 2 * C), ba_ref, 2 * C * c))
        return outs

    y1 = jnp.concatenate(conv_chunked(tb, 0, bar1_ref), axis=1).astype(_CD)
    y2 = jnp.concatenate(conv_chunked(y1, 4 * C, bar2_ref), axis=1)

    o_ref[...] = res + y2


def _chunk_band(m, cin, cout, W, NC):
    N = W * cout
    mp = jnp.pad(m, ((cin, cin), (0, 0)))
    return jnp.stack([
        jnp.concatenate(
            [mp[2 * cin * c:2 * cin * c + 4 * cin,
                d * N + 2 * cout * c:d * N + 2 * cout * (c + 1)]
             for d in range(3)], axis=1)
        for c in range(NC)
    ])


def kernel(input_nchw, skip_nchw, mc1, ba1, m0, ba0, m1, bar1, m2, bar2,
           w11, b11):
    B, Chalf, Hin, Win = input_nchw.shape
    _, _, H, W = skip_nchw.shape
    N = b11.shape[1]
    C = N // W
    NC = W // 2

    bt = 16 if B % 16 == 0 else B
    grid = B // bt
    M = bt * H

    skip2d = skip_nchw.reshape(B, Chalf, H * W).transpose(0, 2, 1).reshape(
        B * H, W * Chalf).astype(_CD)
    xs = input_nchw.reshape(B, Chalf, Hin * Win).transpose(0, 2, 1).reshape(
        B, Hin, Win * Chalf).astype(_CD)
    xv2d = jnp.concatenate([xs, xs], axis=2).reshape(
        B * H, Win * Chalf)

    wall = jnp.concatenate([
        _chunk_band(m1, C, C, W, NC),
        _chunk_band(m2, C, C, W, NC),
        _chunk_band(m0[:W * Chalf], Chalf, C, W, NC),
        _chunk_band(m0[W * Chalf:], Chalf, C, W, NC),
    ], axis=1)

    def full(a):
        return pl.BlockSpec(a.shape, lambda b, n=a.ndim: (0,) * n)

    body = functools.partial(_block_kernel, H=H, C=C, NC=NC, Wn=W, Win=Win)

    out2d = pl.pallas_call(
        body,
        out_shape=jax.ShapeDtypeStruct((B * H, N), jnp.float32),
        grid_spec=pltpu.PrefetchScalarGridSpec(
            num_scalar_prefetch=0,
            grid=(grid,),
            in_specs=[
                pl.BlockSpec((M, W * Chalf), lambda b: (b, 0)),
                pl.BlockSpec((M, Win * Chalf), lambda b: (b, 0)),
                full(wall), full(mc1), full(ba1), full(ba0),
                full(bar1), full(bar2),
                pl.BlockSpec((4 * C, 4 * C), lambda b: (0, 0)),
                full(b11),
            ],
            out_specs=pl.BlockSpec((M, N), lambda b: (b, 0)),
        ),
        compiler_params=pltpu.CompilerParams(
            dimension_semantics=("parallel",)),
    )(skip2d, xv2d, wall, mc1, ba1, ba0, bar1, bar2, w11, b11)

    out = out2d.reshape(B, H, W, C)
    return jnp.transpose(out, (0, 3, 1, 2))
